# baseline (speedup 1.0000x reference)
import jax
import jax.numpy as jnp
from jax import lax
from jax.experimental import pallas as pl
from jax.experimental.pallas import tpu as pltpu
from jax.experimental.pallas import tpu_sc as plsc

N = 10000
E = 320000
D_IN = 128
NP = 10240
NC, NS, L = 2, 16, 16
NW = NC * NS
CK = 128
CHUNKS = 88
E_TILE = CHUNKS * CK
E_PAD = E_TILE * NW
ROWS2D = E_PAD // CK
RB = NP // 4
SLICE = NP // NS
DUMMY = N
GROUP = 4
NGRP = CHUNKS // GROUP
GC = GROUP * CK

f32 = jnp.float32
i32 = jnp.int32


def _dense_first_body(x_ref, w_ref, asv_ref, adv_ref, h_ref, al_ref):
    g = jnp.dot(x_ref[...], w_ref[...].T, preferred_element_type=f32)
    h_ref[...] = g
    a0 = jnp.dot(g, asv_ref[...], preferred_element_type=f32)
    a1 = jnp.dot(g, adv_ref[...], preferred_element_type=f32)
    al_ref[...] = jnp.concatenate([a0, a1], axis=1)


def _dense_first(xp, W, a_src, a_dst, H):
    return pl.pallas_call(
        _dense_first_body,
        grid=(NP // RB,),
        in_specs=[
            pl.BlockSpec((RB, D_IN), lambda i: (i, 0)),
            pl.BlockSpec((H, D_IN), lambda i: (0, 0)),
            pl.BlockSpec((H, 1), lambda i: (0, 0)),
            pl.BlockSpec((H, 1), lambda i: (0, 0)),
        ],
        out_specs=[
            pl.BlockSpec((RB, H), lambda i: (i, 0)),
            pl.BlockSpec((RB, 2), lambda i: (i, 0)),
        ],
        out_shape=[
            jax.ShapeDtypeStruct((NP, H), f32),
            jax.ShapeDtypeStruct((NP, 2), f32),
        ],
    )(xp, W, a_src.reshape(H, 1), a_dst.reshape(H, 1))


def _combine_body(o_ref, d_ref, b_ref, w_ref, asv_ref, adv_ref,
                  hn_ref, al_ref):
    num = o_ref[0] + o_ref[1]
    den = jnp.maximum(d_ref[0] + d_ref[1], 1e-30)
    f = jnp.maximum(num / den + b_ref[...], 0.0)
    g = jnp.dot(f, w_ref[...].T, preferred_element_type=f32)
    hn_ref[...] = g
    a0 = jnp.dot(g, asv_ref[...], preferred_element_type=f32)
    a1 = jnp.dot(g, adv_ref[...], preferred_element_type=f32)
    al_ref[...] = jnp.concatenate([a0, a1], axis=1)


def _combine(o_parts, d_parts, b, W, a_src, a_dst, Hp, Hn):
    return pl.pallas_call(
        _combine_body,
        grid=(NP // RB,),
        in_specs=[
            pl.BlockSpec((2, RB, Hp), lambda i: (0, i, 0)),
            pl.BlockSpec((2, RB, 1), lambda i: (0, i, 0)),
            pl.BlockSpec((1, Hp), lambda i: (0, 0)),
            pl.BlockSpec((Hn, Hp), lambda i: (0, 0)),
            pl.BlockSpec((Hn, 1), lambda i: (0, 0)),
            pl.BlockSpec((Hn, 1), lambda i: (0, 0)),
        ],
        out_specs=[
            pl.BlockSpec((RB, Hn), lambda i: (i, 0)),
            pl.BlockSpec((RB, 2), lambda i: (i, 0)),
        ],
        out_shape=[
            jax.ShapeDtypeStruct((NP, Hn), f32),
            jax.ShapeDtypeStruct((NP, 2), f32),
        ],
    )(o_parts, d_parts.reshape(2, NP, 1), b.reshape(1, Hp),
      W, a_src.reshape(Hn, 1), a_dst.reshape(Hn, 1))


def _final_body(o_ref, d_ref, b_ref, wc_ref, bc_ref, hf_ref, out_ref):
    num = o_ref[0] + o_ref[1]
    den = jnp.maximum(d_ref[0] + d_ref[1], 1e-30)
    hf = jnp.maximum(num / den + b_ref[...], 0.0)
    hf_ref[...] = hf
    out_ref[...] = (
        jnp.dot(hf, wc_ref[...].T, preferred_element_type=f32) + bc_ref[...])


def _final(o_parts, d_parts, b, Wc, bc, Hp, Hc):
    return pl.pallas_call(
        _final_body,
        grid=(NP // RB,),
        in_specs=[
            pl.BlockSpec((2, RB, Hp), lambda i: (0, i, 0)),
            pl.BlockSpec((2, RB, 1), lambda i: (0, i, 0)),
            pl.BlockSpec((1, Hp), lambda i: (0, 0)),
            pl.BlockSpec((Hc, Hp), lambda i: (0, 0)),
            pl.BlockSpec((1, Hc), lambda i: (0, 0)),
        ],
        out_specs=[
            pl.BlockSpec((RB, Hp), lambda i: (i, 0)),
            pl.BlockSpec((RB, Hc), lambda i: (i, 0)),
        ],
        out_shape=[
            jax.ShapeDtypeStruct((NP, Hp), f32),
            jax.ShapeDtypeStruct((NP, Hc), f32),
        ],
    )(o_parts, d_parts.reshape(2, NP, 1), b.reshape(1, Hp),
      Wc, bc.reshape(1, Hc))


def _make_sc_edge(H):
    mesh = plsc.VectorSubcoreMesh(core_axis_name="c", subcore_axis_name="s")

    def body(src_hbm, dst_hbm, as_hbm, ad_hbm, h_hbm, z2_hbm, z1_hbm,
             o_hbm, d_hbm,
             out_sp, den_sp, src_t, dst_t, as_t, ad_t,
             rows0, rows1, rows2, pb0, pb1, pb2,
             sG0, sG1, sG2, sS0, sS1, sS2, sD0, sD1, sD2):
        cid = lax.axis_index("c")
        sid = lax.axis_index("s")
        w = sid * NC + cid
        base = sid * SLICE
        pltpu.sync_copy(z2_hbm.at[pl.ds(base, SLICE)],
                        out_sp.at[pl.ds(base, SLICE)])
        pltpu.sync_copy(z1_hbm.at[pl.ds(base, SLICE)],
                        den_sp.at[pl.ds(base, SLICE)])
        pltpu.sync_copy(as_hbm, as_t)
        pltpu.sync_copy(ad_hbm, ad_t)
        row0 = w * CHUNKS
        pltpu.sync_copy(src_hbm.at[pl.ds(row0, CHUNKS)], src_t)
        pltpu.sync_copy(dst_hbm.at[pl.ds(row0, CHUNKS)], dst_t)
        plsc.subcore_barrier()

        sets = ((rows0, pb0, sG0, sS0, sD0),
                (rows1, pb1, sG1, sS1, sD1),
                (rows2, pb2, sG2, sS2, sD2))

        def fire_gather(g, st):
            rows, _, sG, _, _ = st
            for b in range(GROUP):
                j = g * GROUP + b
                pltpu.make_async_copy(
                    h_hbm.at[src_t.at[j]],
                    rows.at[pl.ds(b * CK, CK)], sG).start()

        def drain_gather(g, st):
            rows, _, sG, _, _ = st
            for b in range(GROUP):
                pltpu.make_async_copy(
                    h_hbm.at[src_t.at[g * GROUP + b]],
                    rows.at[pl.ds(b * CK, CK)], sG).wait()

        def fire_scatter(g, st):
            rows, pb, _, sS, sD = st
            for b in range(GROUP):
                j = g * GROUP + b
                pltpu.make_async_copy(
                    rows.at[pl.ds(b * CK, CK)],
                    out_sp.at[dst_t.at[j]], sS).start(add=True)
                pltpu.make_async_copy(
                    pb.at[pl.ds(b * CK, CK)],
                    den_sp.at[dst_t.at[j]], sD).start(add=True)

        def drain_scatter(g, st):
            rows, pb, _, sS, sD = st
            for b in range(GROUP):
                j = g * GROUP + b
                pltpu.make_async_copy(
                    rows.at[pl.ds(b * CK, CK)],
                    out_sp.at[dst_t.at[j]], sS).wait()
                pltpu.make_async_copy(
                    pb.at[pl.ds(b * CK, CK)],
                    den_sp.at[dst_t.at[j]], sD).wait()

        def compute(g, st, A):
            rows, pb, _, _, _ = st
            for b in range(GROUP):
                j = g * GROUP + b
                for i in range(CK // L):
                    is_v = src_t[j, pl.ds(i * L, L)]
                    id_v = dst_t[j, pl.ds(i * L, L)]
                    as_v = plsc.load_gather(as_t, [is_v])
                    ad_v = plsc.load_gather(ad_t, [id_v])
                    s = as_v + ad_v
                    e = jnp.maximum(s, 0.2 * s)
                    t = A + ad_v
                    c = jnp.maximum(t, 0.2 * t)
                    pb[pl.ds(b * CK + i * L, L)] = jnp.exp(e - c)

            def scale(k, c2):
                k4 = k * 4
                for q in range(4):
                    pk = plsc.load_gather(
                        pb, [jnp.full((L,), q, i32) + k4])
                    for u in range(H // L):
                        rows[k4 + q, pl.ds(u * L, L)] = (
                            rows[k4 + q, pl.ds(u * L, L)] * pk)
                return c2

            lax.fori_loop(0, GC // 4, scale, 0)

        def group_step(g, xi, drain_prev, fire_next):
            X = sets[xi]
            Y = sets[(xi + 1) % 3]
            drain_gather(g, X)
            if drain_prev:
                drain_scatter(g - 2, Y)
            if fire_next:
                fire_gather(g + 1, Y)
            compute(g, X, A)
            fire_scatter(g, X)

        fire_gather(0, sets[0])

        def mred(k, m):
            return jnp.maximum(m, as_t[pl.ds(k * L, L)])
        mv = lax.fori_loop(0, NP // L, mred, jnp.full((L,), -3e38, f32))
        A = lax.reduce_max(mv, (0,))

        group_step(0, 0, False, True)
        group_step(1, 1, False, True)

        def pair3(k, carry):
            g0 = 2 + k * 3

            def gs(g, xi):
                X = sets[xi]
                Y = sets[(xi + 1) % 3]
                drain_gather(g, X)
                drain_scatter(g - 2, Y)

                @pl.when(g + 1 < NGRP)
                def _():
                    fire_gather(g + 1, Y)

                compute(g, X, A)
                fire_scatter(g, X)

            gs(g0, 2)
            gs(g0 + 1, 0)
            gs(g0 + 2, 1)
            return carry

        lax.fori_loop(0, (NGRP - 4) // 3, pair3, 0)

        group_step(NGRP - 2, (NGRP - 2) % 3, True, True)
        Xl = sets[(NGRP - 1) % 3]
        drain_gather(NGRP - 1, Xl)
        drain_scatter(NGRP - 3, sets[(NGRP) % 3])
        compute(NGRP - 1, Xl, A)
        fire_scatter(NGRP - 1, Xl)

        drain_scatter(NGRP - 2, sets[(NGRP - 2) % 3])
        drain_scatter(NGRP - 1, Xl)

        plsc.subcore_barrier()
        pltpu.sync_copy(out_sp.at[pl.ds(base, SLICE)],
                        o_hbm.at[cid, pl.ds(base, SLICE)])
        pltpu.sync_copy(den_sp.at[pl.ds(base, SLICE)],
                        d_hbm.at[cid, pl.ds(base, SLICE)])

    return pl.kernel(
        body,
        out_type=[
            jax.ShapeDtypeStruct((NC, NP, H), f32),
            jax.ShapeDtypeStruct((NC, NP), f32),
        ],
        mesh=mesh,
        compiler_params=pltpu.CompilerParams(
            needs_layout_passes=False, use_tc_tiling_on_sc=False),
        scratch_types=[
            pltpu.VMEM_SHARED((NP, H), f32),
            pltpu.VMEM_SHARED((NP,), f32),
            pltpu.VMEM((CHUNKS, CK), i32),
            pltpu.VMEM((CHUNKS, CK), i32),
            pltpu.VMEM((NP,), f32),
            pltpu.VMEM((NP,), f32),
            pltpu.VMEM((GC, H), f32),
            pltpu.VMEM((GC, H), f32),
            pltpu.VMEM((GC, H), f32),
            pltpu.VMEM((GC,), f32),
            pltpu.VMEM((GC,), f32),
            pltpu.VMEM((GC,), f32),
        ] + [pltpu.SemaphoreType.DMA] * 9,
    )


_sc_edge_16 = _make_sc_edge(16)
_sc_edge_32 = _make_sc_edge(32)


def _sc_edge(H, srcp, dstp, al, h, z2, z1):
    fn = _sc_edge_16 if H == 16 else _sc_edge_32
    as_f = al[:, 0].reshape(NP)
    ad_f = al[:, 1].reshape(NP)
    return fn(srcp, dstp, as_f, ad_f, h, z2[:, :H], z1)


def kernel(x, edge_index, W1, a_src1, a_dst1, b1, W2, a_src2, a_dst2, b2,
           W3, a_src3, a_dst3, b3, Wc, bc):
    src = edge_index[0]
    dst = edge_index[1]
    loop = jnp.arange(N, dtype=i32)
    pad = E_PAD - E - N
    srcp = jnp.concatenate([src, loop, jnp.full((pad,), DUMMY, i32)]).reshape(
        ROWS2D, CK)
    dstp = jnp.concatenate([dst, loop, jnp.full((pad,), DUMMY, i32)]).reshape(
        ROWS2D, CK)
    xp = jnp.pad(x, ((0, NP - N), (0, 0)))
    z2 = jnp.zeros((NP, 32), f32)
    z1 = jnp.zeros((NP,), f32)

    h1, al1 = _dense_first(xp, W1, a_src1, a_dst1, 16)
    o1, d1 = _sc_edge(16, srcp, dstp, al1, h1, z2, z1)

    h2, al2 = _combine(o1, d1, b1, W2, a_src2, a_dst2, 16, 32)
    o2, d2 = _sc_edge(32, srcp, dstp, al2, h2, z2, z1)

    h3, al3 = _combine(o2, d2, b2, W3, a_src3, a_dst3, 32, 32)
    o3, d3 = _sc_edge(32, srcp, dstp, al3, h3, z2, z1)

    hf, logits = _final(o3, d3, b3, Wc, bc, 32, 32)
    return (logits[:N], hf[:N])


# speedup vs baseline: 1.1663x; 1.1663x over previous
import jax
import jax.numpy as jnp
from jax import lax
from jax.experimental import pallas as pl
from jax.experimental.pallas import tpu as pltpu
from jax.experimental.pallas import tpu_sc as plsc

N = 10000
E = 320000
D_IN = 128
NP = 10240
NC, NS, L = 2, 16, 16
NW = NC * NS
CK = 128
CHUNKS = 88
E_TILE = CHUNKS * CK
E_PAD = E_TILE * NW
ROWS2D = E_PAD // CK
RB = NP // 4
SLICE = NP // NS
DUMMY = N
GROUP = 4
NGRP = CHUNKS // GROUP
GC = GROUP * CK

f32 = jnp.float32
i32 = jnp.int32


def _dense_first_body(x_ref, w_ref, asv_ref, adv_ref, h_ref, al_ref):
    g = jnp.dot(x_ref[...], w_ref[...].T, preferred_element_type=f32)
    h_ref[...] = g
    a0 = jnp.dot(g, asv_ref[...], preferred_element_type=f32)
    a1 = jnp.dot(g, adv_ref[...], preferred_element_type=f32)
    al_ref[...] = jnp.concatenate([a0, a1], axis=1)


def _dense_first(xp, W, a_src, a_dst, H):
    return pl.pallas_call(
        _dense_first_body,
        grid=(NP // RB,),
        in_specs=[
            pl.BlockSpec((RB, D_IN), lambda i: (i, 0)),
            pl.BlockSpec((H, D_IN), lambda i: (0, 0)),
            pl.BlockSpec((H, 1), lambda i: (0, 0)),
            pl.BlockSpec((H, 1), lambda i: (0, 0)),
        ],
        out_specs=[
            pl.BlockSpec((RB, H), lambda i: (i, 0)),
            pl.BlockSpec((RB, 2), lambda i: (i, 0)),
        ],
        out_shape=[
            jax.ShapeDtypeStruct((NP, H), f32),
            jax.ShapeDtypeStruct((NP, 2), f32),
        ],
    )(xp, W, a_src.reshape(H, 1), a_dst.reshape(H, 1))


def _combine_body(o_ref, d_ref, b_ref, w_ref, asv_ref, adv_ref,
                  f_ref, hn_ref, al_ref):
    num = o_ref[0] + o_ref[1]
    den = jnp.maximum(d_ref[0] + d_ref[1], 1e-30)
    f = jnp.maximum(num / den + b_ref[...], 0.0)
    f_ref[...] = f
    g = jnp.dot(f, w_ref[...].T, preferred_element_type=f32)
    hn_ref[...] = g
    a0 = jnp.dot(g, asv_ref[...], preferred_element_type=f32)
    a1 = jnp.dot(g, adv_ref[...], preferred_element_type=f32)
    al_ref[...] = jnp.concatenate([a0, a1], axis=1)


def _combine(o_parts, d_parts, b, W, a_src, a_dst, Hp, Hn):
    return pl.pallas_call(
        _combine_body,
        grid=(NP // RB,),
        in_specs=[
            pl.BlockSpec((2, RB, Hp), lambda i: (0, i, 0)),
            pl.BlockSpec((2, RB, 1), lambda i: (0, i, 0)),
            pl.BlockSpec((1, Hp), lambda i: (0, 0)),
            pl.BlockSpec((Hn, Hp), lambda i: (0, 0)),
            pl.BlockSpec((Hn, 1), lambda i: (0, 0)),
            pl.BlockSpec((Hn, 1), lambda i: (0, 0)),
        ],
        out_specs=[
            pl.BlockSpec((RB, Hp), lambda i: (i, 0)),
            pl.BlockSpec((RB, Hn), lambda i: (i, 0)),
            pl.BlockSpec((RB, 2), lambda i: (i, 0)),
        ],
        out_shape=[
            jax.ShapeDtypeStruct((NP, Hp), f32),
            jax.ShapeDtypeStruct((NP, Hn), f32),
            jax.ShapeDtypeStruct((NP, 2), f32),
        ],
    )(o_parts, d_parts.reshape(2, NP, 1), b.reshape(1, Hp),
      W, a_src.reshape(Hn, 1), a_dst.reshape(Hn, 1))


def _combine_deferred_body(o_ref, d_ref, bp_ref, wp_ref, w_ref,
                           asv_ref, adv_ref, f_ref, hn_ref, al_ref):
    num = o_ref[0] + o_ref[1]
    den = jnp.maximum(d_ref[0] + d_ref[1], 1e-30)
    agg = num / den
    gat = jnp.dot(agg, wp_ref[...].T, preferred_element_type=f32)
    f = jnp.maximum(gat + bp_ref[...], 0.0)
    f_ref[...] = f
    g = jnp.dot(f, w_ref[...].T, preferred_element_type=f32)
    hn_ref[...] = g
    a0 = jnp.dot(g, asv_ref[...], preferred_element_type=f32)
    a1 = jnp.dot(g, adv_ref[...], preferred_element_type=f32)
    al_ref[...] = jnp.concatenate([a0, a1], axis=1)


def _combine_deferred(o_parts, d_parts, bp, Wp, W, a_src, a_dst,
                      Hf, Hp, Hn):
    return pl.pallas_call(
        _combine_deferred_body,
        grid=(NP // RB,),
        in_specs=[
            pl.BlockSpec((2, RB, Hf), lambda i: (0, i, 0)),
            pl.BlockSpec((2, RB, 1), lambda i: (0, i, 0)),
            pl.BlockSpec((1, Hp), lambda i: (0, 0)),
            pl.BlockSpec((Hp, Hf), lambda i: (0, 0)),
            pl.BlockSpec((Hn, Hp), lambda i: (0, 0)),
            pl.BlockSpec((Hn, 1), lambda i: (0, 0)),
            pl.BlockSpec((Hn, 1), lambda i: (0, 0)),
        ],
        out_specs=[
            pl.BlockSpec((RB, Hp), lambda i: (i, 0)),
            pl.BlockSpec((RB, Hn), lambda i: (i, 0)),
            pl.BlockSpec((RB, 2), lambda i: (i, 0)),
        ],
        out_shape=[
            jax.ShapeDtypeStruct((NP, Hp), f32),
            jax.ShapeDtypeStruct((NP, Hn), f32),
            jax.ShapeDtypeStruct((NP, 2), f32),
        ],
    )(o_parts, d_parts.reshape(2, NP, 1), bp.reshape(1, Hp), Wp,
      W, a_src.reshape(Hn, 1), a_dst.reshape(Hn, 1))


def _final_body(o_ref, d_ref, b_ref, wc_ref, bc_ref, hf_ref, out_ref):
    num = o_ref[0] + o_ref[1]
    den = jnp.maximum(d_ref[0] + d_ref[1], 1e-30)
    hf = jnp.maximum(num / den + b_ref[...], 0.0)
    hf_ref[...] = hf
    out_ref[...] = (
        jnp.dot(hf, wc_ref[...].T, preferred_element_type=f32) + bc_ref[...])


def _final(o_parts, d_parts, b, Wc, bc, Hp, Hc):
    return pl.pallas_call(
        _final_body,
        grid=(NP // RB,),
        in_specs=[
            pl.BlockSpec((2, RB, Hp), lambda i: (0, i, 0)),
            pl.BlockSpec((2, RB, 1), lambda i: (0, i, 0)),
            pl.BlockSpec((1, Hp), lambda i: (0, 0)),
            pl.BlockSpec((Hc, Hp), lambda i: (0, 0)),
            pl.BlockSpec((1, Hc), lambda i: (0, 0)),
        ],
        out_specs=[
            pl.BlockSpec((RB, Hp), lambda i: (i, 0)),
            pl.BlockSpec((RB, Hc), lambda i: (i, 0)),
        ],
        out_shape=[
            jax.ShapeDtypeStruct((NP, Hp), f32),
            jax.ShapeDtypeStruct((NP, Hc), f32),
        ],
    )(o_parts, d_parts.reshape(2, NP, 1), b.reshape(1, Hp),
      Wc, bc.reshape(1, Hc))


def _make_sc_edge(H):
    mesh = plsc.VectorSubcoreMesh(core_axis_name="c", subcore_axis_name="s")

    def body(src_hbm, dst_hbm, as_hbm, ad_hbm, h_hbm, z2_hbm, z1_hbm,
             o_hbm, d_hbm,
             out_sp, den_sp, src_t, dst_t, as_t, ad_t,
             rows0, rows1, rows2, pb0, pb1, pb2,
             sG0, sG1, sG2, sS0, sS1, sS2, sD0, sD1, sD2):
        cid = lax.axis_index("c")
        sid = lax.axis_index("s")
        w = sid * NC + cid
        base = sid * SLICE
        pltpu.sync_copy(z2_hbm.at[pl.ds(base, SLICE)],
                        out_sp.at[pl.ds(base, SLICE)])
        pltpu.sync_copy(z1_hbm.at[pl.ds(base, SLICE)],
                        den_sp.at[pl.ds(base, SLICE)])
        pltpu.sync_copy(as_hbm, as_t)
        pltpu.sync_copy(ad_hbm, ad_t)
        row0 = w * CHUNKS
        pltpu.sync_copy(src_hbm.at[pl.ds(row0, CHUNKS)], src_t)
        pltpu.sync_copy(dst_hbm.at[pl.ds(row0, CHUNKS)], dst_t)
        plsc.subcore_barrier()

        sets = ((rows0, pb0, sG0, sS0, sD0),
                (rows1, pb1, sG1, sS1, sD1),
                (rows2, pb2, sG2, sS2, sD2))

        def fire_gather(g, st):
            rows, _, sG, _, _ = st
            for b in range(GROUP):
                j = g * GROUP + b
                pltpu.make_async_copy(
                    h_hbm.at[src_t.at[j]],
                    rows.at[pl.ds(b * CK, CK)], sG).start()

        def drain_gather(g, st):
            rows, _, sG, _, _ = st
            for b in range(GROUP):
                pltpu.make_async_copy(
                    h_hbm.at[src_t.at[g * GROUP + b]],
                    rows.at[pl.ds(b * CK, CK)], sG).wait()

        def fire_scatter(g, st):
            rows, pb, _, sS, sD = st
            for b in range(GROUP):
                j = g * GROUP + b
                pltpu.make_async_copy(
                    rows.at[pl.ds(b * CK, CK)],
                    out_sp.at[dst_t.at[j]], sS).start(add=True)
                pltpu.make_async_copy(
                    pb.at[pl.ds(b * CK, CK)],
                    den_sp.at[dst_t.at[j]], sD).start(add=True)

        def drain_scatter(g, st):
            rows, pb, _, sS, sD = st
            for b in range(GROUP):
                j = g * GROUP + b
                pltpu.make_async_copy(
                    rows.at[pl.ds(b * CK, CK)],
                    out_sp.at[dst_t.at[j]], sS).wait()
                pltpu.make_async_copy(
                    pb.at[pl.ds(b * CK, CK)],
                    den_sp.at[dst_t.at[j]], sD).wait()

        def compute(g, st, A):
            rows, pb, _, _, _ = st
            for b in range(GROUP):
                j = g * GROUP + b
                for i in range(CK // L):
                    is_v = src_t[j, pl.ds(i * L, L)]
                    id_v = dst_t[j, pl.ds(i * L, L)]
                    as_v = plsc.load_gather(as_t, [is_v])
                    ad_v = plsc.load_gather(ad_t, [id_v])
                    s = as_v + ad_v
                    e = jnp.maximum(s, 0.2 * s)
                    t = A + ad_v
                    c = jnp.maximum(t, 0.2 * t)
                    pb[pl.ds(b * CK + i * L, L)] = jnp.exp(e - c)

            def scale(k, c2):
                k4 = k * 4
                for q in range(4):
                    pk = plsc.load_gather(
                        pb, [jnp.full((L,), q, i32) + k4])
                    for u in range(H // L):
                        rows[k4 + q, pl.ds(u * L, L)] = (
                            rows[k4 + q, pl.ds(u * L, L)] * pk)
                return c2

            lax.fori_loop(0, GC // 4, scale, 0)

        def group_step(g, xi, drain_prev, fire_next):
            X = sets[xi]
            Y = sets[(xi + 1) % 3]
            drain_gather(g, X)
            if drain_prev:
                drain_scatter(g - 2, Y)
            if fire_next:
                fire_gather(g + 1, Y)
            compute(g, X, A)
            fire_scatter(g, X)

        fire_gather(0, sets[0])

        def mred(k, m):
            return jnp.maximum(m, as_t[pl.ds(k * L, L)])
        mv = lax.fori_loop(0, NP // L, mred, jnp.full((L,), -3e38, f32))
        A = lax.reduce_max(mv, (0,))

        group_step(0, 0, False, True)
        group_step(1, 1, False, True)

        def pair3(k, carry):
            g0 = 2 + k * 3

            def gs(g, xi):
                X = sets[xi]
                Y = sets[(xi + 1) % 3]
                drain_gather(g, X)
                drain_scatter(g - 2, Y)

                @pl.when(g + 1 < NGRP)
                def _():
                    fire_gather(g + 1, Y)

                compute(g, X, A)
                fire_scatter(g, X)

            gs(g0, 2)
            gs(g0 + 1, 0)
            gs(g0 + 2, 1)
            return carry

        lax.fori_loop(0, (NGRP - 4) // 3, pair3, 0)

        group_step(NGRP - 2, (NGRP - 2) % 3, True, True)
        Xl = sets[(NGRP - 1) % 3]
        drain_gather(NGRP - 1, Xl)
        drain_scatter(NGRP - 3, sets[(NGRP) % 3])
        compute(NGRP - 1, Xl, A)
        fire_scatter(NGRP - 1, Xl)

        drain_scatter(NGRP - 2, sets[(NGRP - 2) % 3])
        drain_scatter(NGRP - 1, Xl)

        plsc.subcore_barrier()
        pltpu.sync_copy(out_sp.at[pl.ds(base, SLICE)],
                        o_hbm.at[cid, pl.ds(base, SLICE)])
        pltpu.sync_copy(den_sp.at[pl.ds(base, SLICE)],
                        d_hbm.at[cid, pl.ds(base, SLICE)])

    return pl.kernel(
        body,
        out_type=[
            jax.ShapeDtypeStruct((NC, NP, H), f32),
            jax.ShapeDtypeStruct((NC, NP), f32),
        ],
        mesh=mesh,
        compiler_params=pltpu.CompilerParams(
            needs_layout_passes=False, use_tc_tiling_on_sc=False),
        scratch_types=[
            pltpu.VMEM_SHARED((NP, H), f32),
            pltpu.VMEM_SHARED((NP,), f32),
            pltpu.VMEM((CHUNKS, CK), i32),
            pltpu.VMEM((CHUNKS, CK), i32),
            pltpu.VMEM((NP,), f32),
            pltpu.VMEM((NP,), f32),
            pltpu.VMEM((GC, H), f32),
            pltpu.VMEM((GC, H), f32),
            pltpu.VMEM((GC, H), f32),
            pltpu.VMEM((GC,), f32),
            pltpu.VMEM((GC,), f32),
            pltpu.VMEM((GC,), f32),
        ] + [pltpu.SemaphoreType.DMA] * 9,
    )


_sc_edge_16 = _make_sc_edge(16)
_sc_edge_32 = _make_sc_edge(32)


def _sc_edge(H, srcp, dstp, al, h, z2, z1):
    fn = _sc_edge_16 if H == 16 else _sc_edge_32
    as_f = al[:, 0].reshape(NP)
    ad_f = al[:, 1].reshape(NP)
    return fn(srcp, dstp, as_f, ad_f, h, z2[:, :H], z1)


def kernel(x, edge_index, W1, a_src1, a_dst1, b1, W2, a_src2, a_dst2, b2,
           W3, a_src3, a_dst3, b3, Wc, bc):
    src = edge_index[0]
    dst = edge_index[1]
    loop = jnp.arange(N, dtype=i32)
    pad = E_PAD - E - N
    srcp = jnp.concatenate([src, loop, jnp.full((pad,), DUMMY, i32)]).reshape(
        ROWS2D, CK)
    dstp = jnp.concatenate([dst, loop, jnp.full((pad,), DUMMY, i32)]).reshape(
        ROWS2D, CK)
    xp = jnp.pad(x, ((0, NP - N), (0, 0)))
    z2 = jnp.zeros((NP, 32), f32)
    z1 = jnp.zeros((NP,), f32)

    h1, al1 = _dense_first(xp, W1, a_src1, a_dst1, 16)
    o1, d1 = _sc_edge(16, srcp, dstp, al1, h1, z2, z1)

    f2, h2, al2 = _combine(o1, d1, b1, W2, a_src2, a_dst2, 16, 32)
    o2, d2 = _sc_edge(16, srcp, dstp, al2, f2, z2, z1)

    _, h3, al3 = _combine_deferred(o2, d2, b2, W2, W3, a_src3, a_dst3,
                                   16, 32, 32)
    o3, d3 = _sc_edge(32, srcp, dstp, al3, h3, z2, z1)

    hf, logits = _final(o3, d3, b3, Wc, bc, 32, 32)
    return (logits[:N], hf[:N])


# speedup vs baseline: 1.1691x; 1.0024x over previous
import jax
import jax.numpy as jnp
from jax import lax
from jax.experimental import pallas as pl
from jax.experimental.pallas import tpu as pltpu
from jax.experimental.pallas import tpu_sc as plsc

N = 10000
E = 320000
D_IN = 128
NP = 10240
NC, NS, L = 2, 16, 16
NW = NC * NS
CK = 128
CHUNKS = 88
E_TILE = CHUNKS * CK
E_PAD = E_TILE * NW
ROWS2D = E_PAD // CK
RB = NP // 4
SLICE = NP // NS
DUMMY = N
GROUP = 4
NGRP = CHUNKS // GROUP
GC = GROUP * CK

f32 = jnp.float32
i32 = jnp.int32


def _dense_first_body(x_ref, w_ref, asv_ref, adv_ref, h_ref, al_ref):
    g = jnp.dot(x_ref[...], w_ref[...].T, preferred_element_type=f32)
    h_ref[...] = g
    a0 = jnp.dot(g, asv_ref[...], preferred_element_type=f32)
    a1 = jnp.dot(g, adv_ref[...], preferred_element_type=f32)
    al_ref[...] = jnp.concatenate([a0, a1], axis=1)


def _dense_first(xp, W, a_src, a_dst, H):
    return pl.pallas_call(
        _dense_first_body,
        grid=(NP // RB,),
        in_specs=[
            pl.BlockSpec((RB, D_IN), lambda i: (i, 0)),
            pl.BlockSpec((H, D_IN), lambda i: (0, 0)),
            pl.BlockSpec((H, 1), lambda i: (0, 0)),
            pl.BlockSpec((H, 1), lambda i: (0, 0)),
        ],
        out_specs=[
            pl.BlockSpec((RB, H), lambda i: (i, 0)),
            pl.BlockSpec((RB, 2), lambda i: (i, 0)),
        ],
        out_shape=[
            jax.ShapeDtypeStruct((NP, H), f32),
            jax.ShapeDtypeStruct((NP, 2), f32),
        ],
    )(xp, W, a_src.reshape(H, 1), a_dst.reshape(H, 1))


def _combine_body(o_ref, d_ref, b_ref, w_ref, asv_ref, adv_ref,
                  f_ref, hn_ref, al_ref):
    num = o_ref[0] + o_ref[1]
    den = jnp.maximum(d_ref[0] + d_ref[1], 1e-30)
    f = jnp.maximum(num / den + b_ref[...], 0.0)
    f_ref[...] = f
    g = jnp.dot(f, w_ref[...].T, preferred_element_type=f32)
    hn_ref[...] = g
    a0 = jnp.dot(g, asv_ref[...], preferred_element_type=f32)
    a1 = jnp.dot(g, adv_ref[...], preferred_element_type=f32)
    al_ref[...] = jnp.concatenate([a0, a1], axis=1)


def _combine(o_parts, d_parts, b, W, a_src, a_dst, Hp, Hn):
    return pl.pallas_call(
        _combine_body,
        grid=(NP // RB,),
        in_specs=[
            pl.BlockSpec((2, RB, Hp), lambda i: (0, i, 0)),
            pl.BlockSpec((2, RB, 1), lambda i: (0, i, 0)),
            pl.BlockSpec((1, Hp), lambda i: (0, 0)),
            pl.BlockSpec((Hn, Hp), lambda i: (0, 0)),
            pl.BlockSpec((Hn, 1), lambda i: (0, 0)),
            pl.BlockSpec((Hn, 1), lambda i: (0, 0)),
        ],
        out_specs=[
            pl.BlockSpec((RB, Hp), lambda i: (i, 0)),
            pl.BlockSpec((RB, Hn), lambda i: (i, 0)),
            pl.BlockSpec((RB, 2), lambda i: (i, 0)),
        ],
        out_shape=[
            jax.ShapeDtypeStruct((NP, Hp), f32),
            jax.ShapeDtypeStruct((NP, Hn), f32),
            jax.ShapeDtypeStruct((NP, 2), f32),
        ],
    )(o_parts, d_parts.reshape(2, NP, 1), b.reshape(1, Hp),
      W, a_src.reshape(Hn, 1), a_dst.reshape(Hn, 1))


def _combine_deferred_body(o_ref, d_ref, bp_ref, wp_ref, w_ref,
                           asv_ref, adv_ref, hn_ref, al_ref):
    num = o_ref[0] + o_ref[1]
    den = jnp.maximum(d_ref[0] + d_ref[1], 1e-30)
    agg = num / den
    gat = jnp.dot(agg, wp_ref[...].T, preferred_element_type=f32)
    f = jnp.maximum(gat + bp_ref[...], 0.0)
    g = jnp.dot(f, w_ref[...].T, preferred_element_type=f32)
    hn_ref[...] = g
    a0 = jnp.dot(g, asv_ref[...], preferred_element_type=f32)
    a1 = jnp.dot(g, adv_ref[...], preferred_element_type=f32)
    al_ref[...] = jnp.concatenate([a0, a1], axis=1)


def _combine_deferred(o_parts, d_parts, bp, Wp, W, a_src, a_dst,
                      Hf, Hp, Hn):
    return pl.pallas_call(
        _combine_deferred_body,
        grid=(NP // RB,),
        in_specs=[
            pl.BlockSpec((2, RB, Hf), lambda i: (0, i, 0)),
            pl.BlockSpec((2, RB, 1), lambda i: (0, i, 0)),
            pl.BlockSpec((1, Hp), lambda i: (0, 0)),
            pl.BlockSpec((Hp, Hf), lambda i: (0, 0)),
            pl.BlockSpec((Hn, Hp), lambda i: (0, 0)),
            pl.BlockSpec((Hn, 1), lambda i: (0, 0)),
            pl.BlockSpec((Hn, 1), lambda i: (0, 0)),
        ],
        out_specs=[
            pl.BlockSpec((RB, Hn), lambda i: (i, 0)),
            pl.BlockSpec((RB, 2), lambda i: (i, 0)),
        ],
        out_shape=[
            jax.ShapeDtypeStruct((NP, Hn), f32),
            jax.ShapeDtypeStruct((NP, 2), f32),
        ],
    )(o_parts, d_parts.reshape(2, NP, 1), bp.reshape(1, Hp), Wp,
      W, a_src.reshape(Hn, 1), a_dst.reshape(Hn, 1))


def _final_body(o_ref, d_ref, b_ref, wc_ref, bc_ref, hf_ref, out_ref):
    num = o_ref[0] + o_ref[1]
    den = jnp.maximum(d_ref[0] + d_ref[1], 1e-30)
    hf = jnp.maximum(num / den + b_ref[...], 0.0)
    hf_ref[...] = hf
    out_ref[...] = (
        jnp.dot(hf, wc_ref[...].T, preferred_element_type=f32) + bc_ref[...])


def _final(o_parts, d_parts, b, Wc, bc, Hp, Hc):
    return pl.pallas_call(
        _final_body,
        grid=(NP // RB,),
        in_specs=[
            pl.BlockSpec((2, RB, Hp), lambda i: (0, i, 0)),
            pl.BlockSpec((2, RB, 1), lambda i: (0, i, 0)),
            pl.BlockSpec((1, Hp), lambda i: (0, 0)),
            pl.BlockSpec((Hc, Hp), lambda i: (0, 0)),
            pl.BlockSpec((1, Hc), lambda i: (0, 0)),
        ],
        out_specs=[
            pl.BlockSpec((RB, Hp), lambda i: (i, 0)),
            pl.BlockSpec((RB, Hc), lambda i: (i, 0)),
        ],
        out_shape=[
            jax.ShapeDtypeStruct((NP, Hp), f32),
            jax.ShapeDtypeStruct((NP, Hc), f32),
        ],
    )(o_parts, d_parts.reshape(2, NP, 1), b.reshape(1, Hp),
      Wc, bc.reshape(1, Hc))


def _make_sc_edge(H):
    mesh = plsc.VectorSubcoreMesh(core_axis_name="c", subcore_axis_name="s")

    def body(src_hbm, dst_hbm, as_hbm, ad_hbm, h_hbm, z2_hbm, z1_hbm,
             o_hbm, d_hbm,
             out_sp, den_sp, src_t, dst_t, as_t, ad_t,
             rows0, rows1, rows2, pb0, pb1, pb2,
             sG0, sG1, sG2, sS0, sS1, sS2, sD0, sD1, sD2):
        cid = lax.axis_index("c")
        sid = lax.axis_index("s")
        w = sid * NC + cid
        base = sid * SLICE
        pltpu.sync_copy(z2_hbm.at[pl.ds(base, SLICE)],
                        out_sp.at[pl.ds(base, SLICE)])
        pltpu.sync_copy(z1_hbm.at[pl.ds(base, SLICE)],
                        den_sp.at[pl.ds(base, SLICE)])
        pltpu.sync_copy(as_hbm, as_t)
        pltpu.sync_copy(ad_hbm, ad_t)
        row0 = w * CHUNKS
        pltpu.sync_copy(src_hbm.at[pl.ds(row0, CHUNKS)], src_t)
        pltpu.sync_copy(dst_hbm.at[pl.ds(row0, CHUNKS)], dst_t)
        plsc.subcore_barrier()

        sets = ((rows0, pb0, sG0, sS0, sD0),
                (rows1, pb1, sG1, sS1, sD1),
                (rows2, pb2, sG2, sS2, sD2))

        def fire_gather(g, st):
            rows, _, sG, _, _ = st
            for b in range(GROUP):
                j = g * GROUP + b
                pltpu.make_async_copy(
                    h_hbm.at[src_t.at[j]],
                    rows.at[pl.ds(b * CK, CK)], sG).start()

        def drain_gather(g, st):
            rows, _, sG, _, _ = st
            for b in range(GROUP):
                pltpu.make_async_copy(
                    h_hbm.at[src_t.at[g * GROUP + b]],
                    rows.at[pl.ds(b * CK, CK)], sG).wait()

        def fire_scatter(g, st):
            rows, pb, _, sS, sD = st
            for b in range(GROUP):
                j = g * GROUP + b
                pltpu.make_async_copy(
                    rows.at[pl.ds(b * CK, CK)],
                    out_sp.at[dst_t.at[j]], sS).start(add=True)
                pltpu.make_async_copy(
                    pb.at[pl.ds(b * CK, CK)],
                    den_sp.at[dst_t.at[j]], sD).start(add=True)

        def drain_scatter(g, st):
            rows, pb, _, sS, sD = st
            for b in range(GROUP):
                j = g * GROUP + b
                pltpu.make_async_copy(
                    rows.at[pl.ds(b * CK, CK)],
                    out_sp.at[dst_t.at[j]], sS).wait()
                pltpu.make_async_copy(
                    pb.at[pl.ds(b * CK, CK)],
                    den_sp.at[dst_t.at[j]], sD).wait()

        def compute(g, st, A):
            rows, pb, _, _, _ = st
            for b in range(GROUP):
                j = g * GROUP + b
                for i in range(CK // L):
                    is_v = src_t[j, pl.ds(i * L, L)]
                    id_v = dst_t[j, pl.ds(i * L, L)]
                    as_v = plsc.load_gather(as_t, [is_v])
                    ad_v = plsc.load_gather(ad_t, [id_v])
                    s = as_v + ad_v
                    e = jnp.maximum(s, 0.2 * s)
                    t = A + ad_v
                    c = jnp.maximum(t, 0.2 * t)
                    pb[pl.ds(b * CK + i * L, L)] = jnp.exp(e - c)

            def scale(k, c2):
                k4 = k * 4
                for q in range(4):
                    pk = plsc.load_gather(
                        pb, [jnp.full((L,), q, i32) + k4])
                    for u in range(H // L):
                        rows[k4 + q, pl.ds(u * L, L)] = (
                            rows[k4 + q, pl.ds(u * L, L)] * pk)
                return c2

            lax.fori_loop(0, GC // 4, scale, 0)

        def group_step(g, xi, drain_prev, fire_next):
            X = sets[xi]
            Y = sets[(xi + 1) % 3]
            drain_gather(g, X)
            if drain_prev:
                drain_scatter(g - 2, Y)
            if fire_next:
                fire_gather(g + 1, Y)
            compute(g, X, A)
            fire_scatter(g, X)

        fire_gather(0, sets[0])

        def mred(k, m):
            return jnp.maximum(m, as_t[pl.ds(k * L, L)])
        mv = lax.fori_loop(0, NP // L, mred, jnp.full((L,), -3e38, f32))
        A = lax.reduce_max(mv, (0,))

        group_step(0, 0, False, True)
        group_step(1, 1, False, True)

        def pair3(k, carry):
            g0 = 2 + k * 3

            def gs(g, xi):
                X = sets[xi]
                Y = sets[(xi + 1) % 3]
                drain_gather(g, X)
                drain_scatter(g - 2, Y)

                @pl.when(g + 1 < NGRP)
                def _():
                    fire_gather(g + 1, Y)

                compute(g, X, A)
                fire_scatter(g, X)

            gs(g0, 2)
            gs(g0 + 1, 0)
            gs(g0 + 2, 1)
            return carry

        lax.fori_loop(0, (NGRP - 4) // 3, pair3, 0)

        group_step(NGRP - 2, (NGRP - 2) % 3, True, True)
        Xl = sets[(NGRP - 1) % 3]
        drain_gather(NGRP - 1, Xl)
        drain_scatter(NGRP - 3, sets[(NGRP) % 3])
        compute(NGRP - 1, Xl, A)
        fire_scatter(NGRP - 1, Xl)

        drain_scatter(NGRP - 2, sets[(NGRP - 2) % 3])
        drain_scatter(NGRP - 1, Xl)

        plsc.subcore_barrier()
        pltpu.sync_copy(out_sp.at[pl.ds(base, SLICE)],
                        o_hbm.at[cid, pl.ds(base, SLICE)])
        pltpu.sync_copy(den_sp.at[pl.ds(base, SLICE)],
                        d_hbm.at[cid, pl.ds(base, SLICE)])

    return pl.kernel(
        body,
        out_type=[
            jax.ShapeDtypeStruct((NC, NP, H), f32),
            jax.ShapeDtypeStruct((NC, NP), f32),
        ],
        mesh=mesh,
        compiler_params=pltpu.CompilerParams(
            needs_layout_passes=False, use_tc_tiling_on_sc=False),
        scratch_types=[
            pltpu.VMEM_SHARED((NP, H), f32),
            pltpu.VMEM_SHARED((NP,), f32),
            pltpu.VMEM((CHUNKS, CK), i32),
            pltpu.VMEM((CHUNKS, CK), i32),
            pltpu.VMEM((NP,), f32),
            pltpu.VMEM((NP,), f32),
            pltpu.VMEM((GC, H), f32),
            pltpu.VMEM((GC, H), f32),
            pltpu.VMEM((GC, H), f32),
            pltpu.VMEM((GC,), f32),
            pltpu.VMEM((GC,), f32),
            pltpu.VMEM((GC,), f32),
        ] + [pltpu.SemaphoreType.DMA] * 9,
    )


_sc_edge_16 = _make_sc_edge(16)
_sc_edge_32 = _make_sc_edge(32)


def _sc_edge(H, srcp, dstp, al, h, z2, z1):
    fn = _sc_edge_16 if H == 16 else _sc_edge_32
    as_f = al[:, 0].reshape(NP)
    ad_f = al[:, 1].reshape(NP)
    return fn(srcp, dstp, as_f, ad_f, h, z2[:, :H], z1)


def kernel(x, edge_index, W1, a_src1, a_dst1, b1, W2, a_src2, a_dst2, b2,
           W3, a_src3, a_dst3, b3, Wc, bc):
    src = edge_index[0]
    dst = edge_index[1]
    loop = jnp.arange(N, dtype=i32)
    pad = E_PAD - E - N
    srcp = jnp.concatenate([src, loop, jnp.full((pad,), DUMMY, i32)]).reshape(
        ROWS2D, CK)
    dstp = jnp.concatenate([dst, loop, jnp.full((pad,), DUMMY, i32)]).reshape(
        ROWS2D, CK)
    xp = jnp.pad(x, ((0, NP - N), (0, 0)))
    z2 = jnp.zeros((NP, 32), f32)
    z1 = jnp.zeros((NP,), f32)

    h1, al1 = _dense_first(xp, W1, a_src1, a_dst1, 16)
    o1, d1 = _sc_edge(16, srcp, dstp, al1, h1, z2, z1)

    f2, h2, al2 = _combine(o1, d1, b1, W2, a_src2, a_dst2, 16, 32)
    o2, d2 = _sc_edge(16, srcp, dstp, al2, f2, z2, z1)

    h3, al3 = _combine_deferred(o2, d2, b2, W2, W3, a_src3, a_dst3,
                                16, 32, 32)
    o3, d3 = _sc_edge(32, srcp, dstp, al3, h3, z2, z1)

    hf, logits = _final(o3, d3, b3, Wc, bc, 32, 32)
    return (logits[:N], hf[:N])
